# baseline (speedup 1.0000x reference)
import functools

import jax
import jax.numpy as jnp
from jax import lax
from jax.experimental import pallas as pl
from jax.experimental.pallas import tpu as pltpu
from jax.experimental.pallas import tpu_sc as plsc

_B = 50000
_D = 128
_NC = 2
_NS = 16
_NW = _NC * _NS
_C = 112
_NCHUNK = 14
_BPW = _C * _NCHUNK
_TAILN = _B - (_NW - 1) * _BPW
_NFULL = _TAILN // _C
_TC = _TAILN - _NFULL * _C
_NBUF = 8
_PRIME = 4

_mesh = plsc.VectorSubcoreMesh(core_axis_name="c", subcore_axis_name="s")


@functools.partial(
    pl.kernel,
    mesh=_mesh,
    out_type=jax.ShapeDtypeStruct((_B, _D), jnp.float32),
    scratch_types=[
        pltpu.VMEM((_BPW,), jnp.int32),
        pltpu.VMEM((_NBUF, _C, _D), jnp.float32),
        pltpu.SemaphoreType.DMA,
        pltpu.SemaphoreType.DMA,
    ],
)
def _gather_sc(idx_hbm, x_hbm, out_hbm, idx_v, rows_v, gsem, wsem):
    wid = lax.axis_index("s") * _NC + lax.axis_index("c")
    base = wid * _BPW
    last = wid == _NW - 1

    @pl.when(jnp.logical_not(last))
    def _():
        pltpu.sync_copy(idx_hbm.at[pl.ds(base, _BPW)], idx_v)

    @pl.when(last)
    def _():
        pltpu.sync_copy(idx_hbm.at[pl.ds(base, _TAILN)],
                        idx_v.at[pl.ds(0, _TAILN)])

    def issue_gather(c):
        b = c % _NBUF
        if c < _NFULL:
            return pltpu.async_copy(
                x_hbm.at[idx_v.at[pl.ds(c * _C, _C)]], rows_v.at[b], gsem)
        if c == _NFULL:
            @pl.when(jnp.logical_not(last))
            def _():
                pltpu.async_copy(
                    x_hbm.at[idx_v.at[pl.ds(c * _C, _C)]], rows_v.at[b], gsem)

            @pl.when(last)
            def _():
                pltpu.async_copy(
                    x_hbm.at[idx_v.at[pl.ds(c * _C, _TC)]],
                    rows_v.at[b, pl.ds(0, _TC)], gsem)
        else:
            @pl.when(jnp.logical_not(last))
            def _():
                pltpu.async_copy(
                    x_hbm.at[idx_v.at[pl.ds(c * _C, _C)]], rows_v.at[b], gsem)
        return None

    gathers = {}
    writes = {}
    for c in range(_PRIME):
        gathers[c] = issue_gather(c)
    for c in range(_NFULL):
        g = c + _PRIME
        if g < _NCHUNK:
            wc = g - _NBUF
            if wc >= 0:
                writes[wc].wait()
            gathers[g] = issue_gather(g)
        b = c % _NBUF
        gathers[c].wait()
        writes[c] = pltpu.async_copy(
            rows_v.at[b], out_hbm.at[pl.ds(base + c * _C, _C)], wsem)

    b12 = _NFULL % _NBUF
    b13 = (_NFULL + 1) % _NBUF

    @pl.when(jnp.logical_not(last))
    def _():
        pltpu.make_async_copy(
            x_hbm.at[idx_v.at[pl.ds(_NFULL * _C, _C)]],
            rows_v.at[b12], gsem).wait()
        pltpu.async_copy(
            rows_v.at[b12], out_hbm.at[pl.ds(base + _NFULL * _C, _C)], wsem)
        pltpu.make_async_copy(
            x_hbm.at[idx_v.at[pl.ds((_NFULL + 1) * _C, _C)]],
            rows_v.at[b13], gsem).wait()
        pltpu.async_copy(
            rows_v.at[b13],
            out_hbm.at[pl.ds(base + (_NFULL + 1) * _C, _C)], wsem)

    @pl.when(last)
    def _():
        pltpu.make_async_copy(
            x_hbm.at[idx_v.at[pl.ds(_NFULL * _C, _TC)]],
            rows_v.at[b12, pl.ds(0, _TC)], gsem).wait()
        pltpu.async_copy(
            rows_v.at[b12, pl.ds(0, _TC)],
            out_hbm.at[pl.ds(base + _NFULL * _C, _TC)], wsem)

    for wc in range(_NCHUNK - _NBUF, _NFULL):
        writes[wc].wait()

    @pl.when(jnp.logical_not(last))
    def _():
        pltpu.make_async_copy(
            rows_v.at[b12], out_hbm.at[pl.ds(base + _NFULL * _C, _C)],
            wsem).wait()
        pltpu.make_async_copy(
            rows_v.at[b13],
            out_hbm.at[pl.ds(base + (_NFULL + 1) * _C, _C)], wsem).wait()

    @pl.when(last)
    def _():
        pltpu.make_async_copy(
            rows_v.at[b12, pl.ds(0, _TC)],
            out_hbm.at[pl.ds(base + _NFULL * _C, _TC)], wsem).wait()


def kernel(indices, x):
    return _gather_sc(jnp.reshape(indices, (_B,)), x)


# speedup vs baseline: 1.0131x; 1.0131x over previous
import functools

import jax
import jax.numpy as jnp
from jax import lax
from jax.experimental import pallas as pl
from jax.experimental.pallas import tpu as pltpu
from jax.experimental.pallas import tpu_sc as plsc

_B = 50000
_D = 128
_NC = 2
_NS = 16
_NW = _NC * _NS
_C = 224
_NCHUNK = 7
_BPW = _C * _NCHUNK
_TAILN = _B - (_NW - 1) * _BPW
_NFULL = _TAILN // _C
_TC = _TAILN - _NFULL * _C
_NBUF = 4
_PRIME = 2

_mesh = plsc.VectorSubcoreMesh(core_axis_name="c", subcore_axis_name="s")


@functools.partial(
    pl.kernel,
    mesh=_mesh,
    out_type=jax.ShapeDtypeStruct((_B, _D), jnp.float32),
    scratch_types=[
        pltpu.VMEM((_BPW,), jnp.int32),
        pltpu.VMEM((_NBUF, _C, _D), jnp.float32),
        pltpu.SemaphoreType.DMA,
        pltpu.SemaphoreType.DMA,
    ],
)
def _gather_sc(idx_hbm, x_hbm, out_hbm, idx_v, rows_v, gsem, wsem):
    wid = lax.axis_index("s") * _NC + lax.axis_index("c")
    base = wid * _BPW
    last = wid == _NW - 1

    @pl.when(jnp.logical_not(last))
    def _():
        pltpu.sync_copy(idx_hbm.at[pl.ds(base, _BPW)], idx_v)

    @pl.when(last)
    def _():
        pltpu.sync_copy(idx_hbm.at[pl.ds(base, _TAILN)],
                        idx_v.at[pl.ds(0, _TAILN)])

    def issue_gather(c):
        b = c % _NBUF
        if c < _NFULL:
            return pltpu.async_copy(
                x_hbm.at[idx_v.at[pl.ds(c * _C, _C)]], rows_v.at[b], gsem)

        @pl.when(jnp.logical_not(last))
        def _():
            pltpu.async_copy(
                x_hbm.at[idx_v.at[pl.ds(c * _C, _C)]], rows_v.at[b], gsem)

        if c == _NFULL:
            @pl.when(last)
            def _():
                pltpu.async_copy(
                    x_hbm.at[idx_v.at[pl.ds(c * _C, _TC)]],
                    rows_v.at[b, pl.ds(0, _TC)], gsem)
        return None

    gathers = {}
    writes = {}
    for c in range(_PRIME):
        gathers[c] = issue_gather(c)
    for c in range(_NFULL):
        g = c + _PRIME
        if g < _NCHUNK:
            wc = g - _NBUF
            if wc >= 0:
                writes[wc].wait()
            gathers[g] = issue_gather(g)
        b = c % _NBUF
        gathers[c].wait()
        writes[c] = pltpu.async_copy(
            rows_v.at[b], out_hbm.at[pl.ds(base + c * _C, _C)], wsem)

    @pl.when(jnp.logical_not(last))
    def _():
        for c in range(_NFULL, _NCHUNK):
            b = c % _NBUF
            pltpu.make_async_copy(
                x_hbm.at[idx_v.at[pl.ds(c * _C, _C)]],
                rows_v.at[b], gsem).wait()
            pltpu.async_copy(
                rows_v.at[b], out_hbm.at[pl.ds(base + c * _C, _C)], wsem)

    @pl.when(last)
    def _():
        b = _NFULL % _NBUF
        pltpu.make_async_copy(
            x_hbm.at[idx_v.at[pl.ds(_NFULL * _C, _TC)]],
            rows_v.at[b, pl.ds(0, _TC)], gsem).wait()
        pltpu.async_copy(
            rows_v.at[b, pl.ds(0, _TC)],
            out_hbm.at[pl.ds(base + _NFULL * _C, _TC)], wsem)

    for wc in range(max(0, _NCHUNK - _NBUF), _NFULL):
        writes[wc].wait()

    @pl.when(jnp.logical_not(last))
    def _():
        for c in range(_NFULL, _NCHUNK):
            b = c % _NBUF
            pltpu.make_async_copy(
                rows_v.at[b], out_hbm.at[pl.ds(base + c * _C, _C)],
                wsem).wait()

    @pl.when(last)
    def _():
        b = _NFULL % _NBUF
        pltpu.make_async_copy(
            rows_v.at[b, pl.ds(0, _TC)],
            out_hbm.at[pl.ds(base + _NFULL * _C, _TC)], wsem).wait()


def kernel(indices, x):
    return _gather_sc(jnp.reshape(indices, (_B,)), x)
